# baseline (speedup 1.0000x reference)
import functools

import jax
import jax.numpy as jnp
import numpy as np
from jax import lax
from jax.experimental import pallas as pl
from jax.experimental.pallas import tpu as pltpu
from jax.experimental.pallas import tpu_sc as plsc

D = 128
NH = 4
ORDERS = (2, 3)
PRIMES = (100003, 100019)
NPART = NH * len(ORDERS)
TOT = D * NPART

NC, NS = 2, 16
NW = NC * NS


def _hash_multipliers():
    np.random.seed(42)
    return [[int(np.random.randint(2, p - 1)) for _ in range(NH)] for p in PRIMES]


MULTS = _hash_multipliers()


def _mod_p(x, p, rp):
    q = (x.astype(jnp.float32) * rp).astype(jnp.int32)
    r = x - q * p
    r = jnp.where(r < 0, r + p, r)
    r = jnp.where(r >= p, r - p, r)
    return r


def _sc_gather_body(t0_hbm, tm1_hbm, tm2_hbm, tbl2_hbm, tbl3_hbm, mem_hbm,
                    tok0_v, tok1_v, tok2_v, idx_v, rows_v, gsems, wsems,
                    n_pos):
    cid = lax.axis_index("c")
    sid = lax.axis_index("s")
    wid = sid * NC + cid
    base = wid * n_pos
    nvec = n_pos // 16

    pltpu.sync_copy(t0_hbm.at[pl.ds(base, n_pos)], tok0_v)
    pltpu.sync_copy(tm1_hbm.at[pl.ds(base, n_pos)], tok1_v)
    pltpu.sync_copy(tm2_hbm.at[pl.ds(base, n_pos)], tok2_v)

    for i in range(nvec):
        j, k = divmod(i, 8)
        t0 = tok0_v[pl.ds(i * 16, 16)]
        t1 = tok1_v[pl.ds(i * 16, 16)]
        t2 = tok2_v[pl.ds(i * 16, 16)]
        for n_idx, n in enumerate(ORDERS):
            p = PRIMES[n_idx]
            rp = np.float32(1.0 / p)
            for h in range(NH):
                m = MULTS[n_idx][h]
                m_hi, m_lo = divmod(m, 512)
                if n == 2:
                    acc = _mod_p(t1 * m_hi, p, rp) * 512 + t1 * m_lo + t0
                    hh = _mod_p(acc, p, rp)
                else:
                    acc = _mod_p(t2 * m_hi, p, rp) * 512 + t2 * m_lo + t1
                    h1 = _mod_p(acc, p, rp)
                    acc2 = _mod_p(h1 * m_hi, p, rp) * 512 + h1 * m_lo + t0
                    hh = _mod_p(acc2, p, rp)
                part = n_idx * NH + h
                idx_v[np.int32(part), np.int32(j), pl.ds(k * 16, 16)] = (
                    hh * NH + h)

    is_row_start = (base % 2048) == 0
    nchunk = n_pos // 128
    NB = 4

    def start_gather(c, b):
        part, j = divmod(c, nchunk)
        tbl = tbl2_hbm if part < NH else tbl3_hbm
        src = tbl.at[idx_v.at[np.int32(part), np.int32(j)]]
        return pltpu.async_copy(src, rows_v.at[np.int32(b)],
                                gsems.at[np.int32(b)])

    def zero_pads(c, b):
        part, j = divmod(c, nchunk)
        if j == 0:
            npad = 1 if part < NH else 2

            @pl.when(is_row_start)
            def _zero_pad_rows():
                zero = jnp.zeros((16,), jnp.float32)
                for r in range(npad):
                    for cc in range(8):
                        rows_v[np.int32(b), np.int32(r),
                               pl.ds(cc * 16, 16)] = zero

    def start_write(c, b):
        part, j = divmod(c, nchunk)
        return pltpu.async_copy(rows_v.at[np.int32(b)],
                                mem_hbm.at[pl.ds(base + j * 128, 128),
                                           pl.ds(part * D, D)],
                                wsems.at[np.int32(b)])

    total = NPART * nchunk
    wh = {}
    for g0 in range(0, total, NB):
        gsz = min(NB, total - g0)
        ghs = []
        for t in range(gsz):
            c = g0 + t
            if c >= NB:
                wh.pop(c - NB).wait()
            ghs.append(start_gather(c, t))
        for t in range(gsz):
            c = g0 + t
            ghs[t].wait()
            zero_pads(c, t)
            wh[c] = start_write(c, t)
    for h in wh.values():
        h.wait()


def _make_sc_gather(bs, n_pos):
    mesh = plsc.VectorSubcoreMesh(core_axis_name="c", subcore_axis_name="s")
    return pl.kernel(
        functools.partial(_sc_gather_body, n_pos=n_pos),
        out_type=jax.ShapeDtypeStruct((bs, TOT), jnp.float32),
        mesh=mesh,
        scratch_types=[
            pltpu.VMEM((n_pos,), jnp.int32),
            pltpu.VMEM((n_pos,), jnp.int32),
            pltpu.VMEM((n_pos,), jnp.int32),
            pltpu.VMEM((NPART, n_pos // 128, 128), jnp.int32),
            pltpu.VMEM((4, 128, D), jnp.float32),
            pltpu.SemaphoreType.DMA((4,)),
            pltpu.SemaphoreType.DMA((4,)),
        ],
    )


def _dense_body(hid_ref, mem_ref, wq_ref, bq_ref, wk_ref, bk_ref,
                wv_ref, bv_ref, wo_ref, bo_ref, out_ref, gw_ref):
    hid = hid_ref[...]
    mem = mem_ref[...]
    q = jnp.dot(hid, wq_ref[...], preferred_element_type=jnp.float32,
                precision=lax.Precision.HIGHEST) + bq_ref[...]
    k = jnp.dot(mem, wk_ref[...], preferred_element_type=jnp.float32,
                precision=lax.Precision.HIGHEST) + bk_ref[...]
    v = jnp.dot(mem, wv_ref[...], preferred_element_type=jnp.float32,
                precision=lax.Precision.HIGHEST) + bv_ref[...]
    s = jnp.sum(q * k, axis=-1, keepdims=True) * np.float32(1.0 / np.sqrt(D))
    g = jax.nn.sigmoid(s)
    out_ref[...] = jnp.dot(g * v, wo_ref[...], preferred_element_type=jnp.float32,
                           precision=lax.Precision.HIGHEST) + bo_ref[...]
    gw_ref[...] = g


def _make_dense(bs, blk, interpret=False):
    grid = (bs // blk,)
    Z = np.int32(0)
    full = lambda i: (Z, Z)
    row = lambda i: (i, Z)
    return pl.pallas_call(
        _dense_body,
        grid=grid,
        in_specs=[
            pl.BlockSpec((blk, D), row),
            pl.BlockSpec((blk, TOT), row),
            pl.BlockSpec((D, D), full),
            pl.BlockSpec((1, D), full),
            pl.BlockSpec((TOT, D), full),
            pl.BlockSpec((1, D), full),
            pl.BlockSpec((TOT, D), full),
            pl.BlockSpec((1, D), full),
            pl.BlockSpec((D, D), full),
            pl.BlockSpec((1, D), full),
        ],
        out_specs=[
            pl.BlockSpec((blk, D), row),
            pl.BlockSpec((blk, 1), row),
        ],
        out_shape=[
            jax.ShapeDtypeStruct((bs, D), jnp.float32),
            jax.ShapeDtypeStruct((bs, 1), jnp.float32),
        ],
        interpret=interpret,
    )


def kernel(token_ids, hidden_states, tables_n2, tables_n3,
           Wq, bq, Wk, bk, Wv, bv, Wo, bo):
    B, S = token_ids.shape
    bs = B * S
    n_pos = bs // NW

    tok = token_ids.astype(jnp.int32)
    t0 = tok.reshape(bs)
    tm1 = jnp.pad(tok, ((0, 0), (1, 0)))[:, :S].reshape(bs)
    tm2 = jnp.pad(tok, ((0, 0), (2, 0)))[:, :S].reshape(bs)

    tbl2v = jnp.transpose(tables_n2, (1, 0, 2)).reshape(PRIMES[0] * NH, D)
    tbl3v = jnp.transpose(tables_n3, (1, 0, 2)).reshape(PRIMES[1] * NH, D)
    mem2 = _make_sc_gather(bs, n_pos)(t0, tm1, tm2, tbl2v, tbl3v)

    hid = hidden_states.reshape(bs, D)
    out2, gw2 = _make_dense(bs, 512)(
        hid, mem2, Wq, bq.reshape(1, D), Wk, bk.reshape(1, D),
        Wv, bv.reshape(1, D), Wo, bo.reshape(1, D))

    return (mem2.reshape(B, S, TOT),
            out2.reshape(B, S, D).astype(jnp.float64),
            gw2.reshape(B, S, 1).astype(jnp.float64))


# speedup vs baseline: 3.5458x; 3.5458x over previous
import functools

import jax
import jax.numpy as jnp
import numpy as np
from jax import lax
from jax.experimental import pallas as pl
from jax.experimental.pallas import tpu as pltpu
from jax.experimental.pallas import tpu_sc as plsc

D = 128
NH = 4
ORDERS = (2, 3)
PRIMES = (100003, 100019)
NPART = NH * len(ORDERS)
TOT = D * NPART

NC, NS = 2, 16
NW = NC * NS


def _hash_multipliers():
    np.random.seed(42)
    return [[int(np.random.randint(2, p - 1)) for _ in range(NH)] for p in PRIMES]


MULTS = _hash_multipliers()


def _mod_p(x, p, rp):
    q = (x.astype(jnp.float32) * rp).astype(jnp.int32)
    r = x - q * p
    r = jnp.where(r < 0, r + p, r)
    r = jnp.where(r >= p, r - p, r)
    return r


def _sc_gather_body(t0_hbm, tm1_hbm, tm2_hbm, tbl2_hbm, tbl3_hbm, mem_hbm,
                    tok0_v, tok1_v, tok2_v, idx_v, rows_v, gsems, wsems,
                    n_pos):
    cid = lax.axis_index("c")
    sid = lax.axis_index("s")
    wid = sid * NC + cid
    base = wid * n_pos
    nvec = n_pos // 16

    pltpu.sync_copy(t0_hbm.at[pl.ds(base, n_pos)], tok0_v)
    pltpu.sync_copy(tm1_hbm.at[pl.ds(base, n_pos)], tok1_v)
    pltpu.sync_copy(tm2_hbm.at[pl.ds(base, n_pos)], tok2_v)

    for i in range(nvec):
        j, k = divmod(i, 8)
        t0 = tok0_v[pl.ds(i * 16, 16)]
        t1 = tok1_v[pl.ds(i * 16, 16)]
        t2 = tok2_v[pl.ds(i * 16, 16)]
        for n_idx, n in enumerate(ORDERS):
            p = PRIMES[n_idx]
            rp = np.float32(1.0 / p)
            for h in range(NH):
                m = MULTS[n_idx][h]
                m_hi, m_lo = divmod(m, 512)
                if n == 2:
                    acc = _mod_p(t1 * m_hi, p, rp) * 512 + t1 * m_lo + t0
                    hh = _mod_p(acc, p, rp)
                else:
                    acc = _mod_p(t2 * m_hi, p, rp) * 512 + t2 * m_lo + t1
                    h1 = _mod_p(acc, p, rp)
                    acc2 = _mod_p(h1 * m_hi, p, rp) * 512 + h1 * m_lo + t0
                    hh = _mod_p(acc2, p, rp)
                part = n_idx * NH + h
                idx_v[np.int32(part), np.int32(j), pl.ds(k * 16, 16)] = (
                    hh * NH + h)

    is_row_start = (base % 2048) == 0
    nchunk = n_pos // 128
    NB = 4

    def start_gather(c, b):
        part, j = divmod(c, nchunk)
        tbl = tbl2_hbm if part < NH else tbl3_hbm
        p = PRIMES[0] if part < NH else PRIMES[1]
        tbl2d = tbl.reshape(p * NH, D)
        src = tbl2d.at[idx_v.at[np.int32(part), np.int32(j)]]
        return pltpu.async_copy(src, rows_v.at[np.int32(b)],
                                gsems.at[np.int32(b)])

    def zero_pads(c, b):
        part, j = divmod(c, nchunk)
        if j == 0:
            npad = 1 if part < NH else 2

            @pl.when(is_row_start)
            def _zero_pad_rows():
                zero = jnp.zeros((16,), jnp.float32)
                for r in range(npad):
                    for cc in range(8):
                        rows_v[np.int32(b), np.int32(r),
                               pl.ds(cc * 16, 16)] = zero

    def start_write(c, b):
        part, j = divmod(c, nchunk)
        return pltpu.async_copy(rows_v.at[np.int32(b)],
                                mem_hbm.at[pl.ds(base + j * 128, 128),
                                           pl.ds(part * D, D)],
                                wsems.at[np.int32(b)])

    total = NPART * nchunk
    wh = {}
    for g0 in range(0, total, NB):
        gsz = min(NB, total - g0)
        ghs = []
        for t in range(gsz):
            c = g0 + t
            if c >= NB:
                wh.pop(c - NB).wait()
            ghs.append(start_gather(c, t))
        for t in range(gsz):
            c = g0 + t
            ghs[t].wait()
            zero_pads(c, t)
            wh[c] = start_write(c, t)
    for h in wh.values():
        h.wait()


def _make_sc_gather(bs, n_pos):
    mesh = plsc.VectorSubcoreMesh(core_axis_name="c", subcore_axis_name="s")
    return pl.kernel(
        functools.partial(_sc_gather_body, n_pos=n_pos),
        out_type=jax.ShapeDtypeStruct((bs, TOT), jnp.float32),
        mesh=mesh,
        scratch_types=[
            pltpu.VMEM((n_pos,), jnp.int32),
            pltpu.VMEM((n_pos,), jnp.int32),
            pltpu.VMEM((n_pos,), jnp.int32),
            pltpu.VMEM((NPART, n_pos // 128, 128), jnp.int32),
            pltpu.VMEM((4, 128, D), jnp.float32),
            pltpu.SemaphoreType.DMA((4,)),
            pltpu.SemaphoreType.DMA((4,)),
        ],
    )


def _dense_body(hid_ref, mem_ref, wq_ref, bq_ref, wk_ref, bk_ref,
                wv_ref, bv_ref, wo_ref, bo_ref, out_ref, gw_ref):
    hid = hid_ref[...]
    mem = mem_ref[...]
    q = jnp.dot(hid, wq_ref[...], preferred_element_type=jnp.float32,
                precision=lax.Precision.HIGHEST) + bq_ref[...]
    k = jnp.dot(mem, wk_ref[...], preferred_element_type=jnp.float32,
                precision=lax.Precision.HIGHEST) + bk_ref[...]
    v = jnp.dot(mem, wv_ref[...], preferred_element_type=jnp.float32,
                precision=lax.Precision.HIGHEST) + bv_ref[...]
    s = jnp.sum(q * k, axis=-1, keepdims=True) * np.float32(1.0 / np.sqrt(D))
    g = jax.nn.sigmoid(s)
    out_ref[...] = jnp.dot(g * v, wo_ref[...], preferred_element_type=jnp.float32,
                           precision=lax.Precision.HIGHEST) + bo_ref[...]
    gw_ref[...] = g


def _make_dense(bs, blk, interpret=False):
    grid = (bs // blk,)
    Z = np.int32(0)
    full = lambda i: (Z, Z)
    row = lambda i: (i, Z)
    return pl.pallas_call(
        _dense_body,
        grid=grid,
        in_specs=[
            pl.BlockSpec((blk, D), row),
            pl.BlockSpec((blk, TOT), row),
            pl.BlockSpec((D, D), full),
            pl.BlockSpec((1, D), full),
            pl.BlockSpec((TOT, D), full),
            pl.BlockSpec((1, D), full),
            pl.BlockSpec((TOT, D), full),
            pl.BlockSpec((1, D), full),
            pl.BlockSpec((D, D), full),
            pl.BlockSpec((1, D), full),
        ],
        out_specs=[
            pl.BlockSpec((blk, D), row),
            pl.BlockSpec((blk, 1), row),
        ],
        out_shape=[
            jax.ShapeDtypeStruct((bs, D), jnp.float32),
            jax.ShapeDtypeStruct((bs, 1), jnp.float32),
        ],
        interpret=interpret,
    )


def kernel(token_ids, hidden_states, tables_n2, tables_n3,
           Wq, bq, Wk, bk, Wv, bv, Wo, bo):
    B, S = token_ids.shape
    bs = B * S
    n_pos = bs // NW

    tok = token_ids.astype(jnp.int32)
    t0 = tok.reshape(bs)
    tm1 = jnp.pad(tok, ((0, 0), (1, 0)))[:, :S].reshape(bs)
    tm2 = jnp.pad(tok, ((0, 0), (2, 0)))[:, :S].reshape(bs)

    tbl2v = jnp.transpose(tables_n2, (1, 0, 2))
    tbl3v = jnp.transpose(tables_n3, (1, 0, 2))
    mem2 = _make_sc_gather(bs, n_pos)(t0, tm1, tm2, tbl2v, tbl3v)

    hid = hidden_states.reshape(bs, D)
    out2, gw2 = _make_dense(bs, 512)(
        hid, mem2, Wq, bq.reshape(1, D), Wk, bk.reshape(1, D),
        Wv, bv.reshape(1, D), Wo, bo.reshape(1, D))

    return (mem2.reshape(B, S, TOT),
            out2.reshape(B, S, D).astype(jnp.float64),
            gw2.reshape(B, S, 1).astype(jnp.float64))


# speedup vs baseline: 4.9531x; 1.3969x over previous
import functools

import jax
import jax.numpy as jnp
import numpy as np
from jax import lax
from jax.experimental import pallas as pl
from jax.experimental.pallas import tpu as pltpu
from jax.experimental.pallas import tpu_sc as plsc

D = 128
NH = 4
ORDERS = (2, 3)
PRIMES = (100003, 100019)
NPART = NH * len(ORDERS)
TOT = D * NPART

NC, NS = 2, 16
NW = NC * NS


def _hash_multipliers():
    np.random.seed(42)
    return [[int(np.random.randint(2, p - 1)) for _ in range(NH)] for p in PRIMES]


MULTS = _hash_multipliers()


def _mod_p(x, p, rp):
    q = (x.astype(jnp.float32) * rp).astype(jnp.int32)
    r = x - q * p
    r = jnp.where(r < 0, r + p, r)
    r = jnp.where(r >= p, r - p, r)
    return r


def _sc_gather_body(t0_hbm, tm1_hbm, tm2_hbm, tbl2_hbm, tbl3_hbm, mem_hbm,
                    tok0_v, tok1_v, tok2_v, idx_v, rows_v, gsems, wsems,
                    n_pos):
    cid = lax.axis_index("c")
    sid = lax.axis_index("s")
    wid = sid * NC + cid
    base = wid * n_pos
    nvec = n_pos // 16

    pltpu.sync_copy(t0_hbm.at[pl.ds(base, n_pos)], tok0_v)
    pltpu.sync_copy(tm1_hbm.at[pl.ds(base, n_pos)], tok1_v)
    pltpu.sync_copy(tm2_hbm.at[pl.ds(base, n_pos)], tok2_v)

    for i in range(nvec):
        j, k = divmod(i, 8)
        t0 = tok0_v[pl.ds(i * 16, 16)]
        t1 = tok1_v[pl.ds(i * 16, 16)]
        t2 = tok2_v[pl.ds(i * 16, 16)]
        for n_idx, n in enumerate(ORDERS):
            p = PRIMES[n_idx]
            rp = np.float32(1.0 / p)
            for h in range(NH):
                m = MULTS[n_idx][h]
                m_hi, m_lo = divmod(m, 512)
                if n == 2:
                    acc = _mod_p(t1 * m_hi, p, rp) * 512 + t1 * m_lo + t0
                    hh = _mod_p(acc, p, rp)
                else:
                    acc = _mod_p(t2 * m_hi, p, rp) * 512 + t2 * m_lo + t1
                    h1 = _mod_p(acc, p, rp)
                    acc2 = _mod_p(h1 * m_hi, p, rp) * 512 + h1 * m_lo + t0
                    hh = _mod_p(acc2, p, rp)
                part = n_idx * NH + h
                idx_v[np.int32(part), np.int32(j), pl.ds(k * 16, 16)] = (
                    hh * NH + h)

    is_row_start = (base % 2048) == 0
    nchunk = n_pos // 128
    NB = 4

    def start_gather(c, b):
        part, j = divmod(c, nchunk)
        tbl = tbl2_hbm if part < NH else tbl3_hbm
        p = PRIMES[0] if part < NH else PRIMES[1]
        tbl2d = tbl.reshape(p * NH, D)
        src = tbl2d.at[idx_v.at[np.int32(part), np.int32(j)]]
        return pltpu.async_copy(src, rows_v.at[np.int32(b)],
                                gsems.at[np.int32(b)])

    def zero_pads(c, b):
        part, j = divmod(c, nchunk)
        if j == 0:
            npad = 1 if part < NH else 2

            @pl.when(is_row_start)
            def _zero_pad_rows():
                zero = jnp.zeros((16,), jnp.float32)
                for r in range(npad):
                    for cc in range(8):
                        rows_v[np.int32(b), np.int32(r),
                               pl.ds(cc * 16, 16)] = zero

    def start_write(c, b):
        part, j = divmod(c, nchunk)
        return pltpu.async_copy(rows_v.at[np.int32(b)],
                                mem_hbm.at[pl.ds(base + j * 128, 128),
                                           pl.ds(part * D, D)],
                                wsems.at[np.int32(b)])

    total = NPART * nchunk
    wh = {}
    for g0 in range(0, total, NB):
        gsz = min(NB, total - g0)
        ghs = []
        for t in range(gsz):
            c = g0 + t
            if c >= NB:
                wh.pop(c - NB).wait()
            ghs.append(start_gather(c, t))
        for t in range(gsz):
            c = g0 + t
            ghs[t].wait()
            zero_pads(c, t)
            wh[c] = start_write(c, t)
    for h in wh.values():
        h.wait()


def _make_sc_gather(bs, n_pos):
    mesh = plsc.VectorSubcoreMesh(core_axis_name="c", subcore_axis_name="s")
    return pl.kernel(
        functools.partial(_sc_gather_body, n_pos=n_pos),
        out_type=jax.ShapeDtypeStruct((bs, TOT), jnp.float32),
        mesh=mesh,
        scratch_types=[
            pltpu.VMEM((n_pos,), jnp.int32),
            pltpu.VMEM((n_pos,), jnp.int32),
            pltpu.VMEM((n_pos,), jnp.int32),
            pltpu.VMEM((NPART, n_pos // 128, 128), jnp.int32),
            pltpu.VMEM((4, 128, D), jnp.float32),
            pltpu.SemaphoreType.DMA((4,)),
            pltpu.SemaphoreType.DMA((4,)),
        ],
    )


def _dense_body(hid_ref, mem_ref, wq_ref, bq_ref, wk_ref, bk_ref,
                wv_ref, bv_ref, wo_ref, bo_ref, out_ref, gw_ref):
    hid = hid_ref[...]
    mem = mem_ref[...]
    q = jnp.dot(hid, wq_ref[...], preferred_element_type=jnp.float32,
                precision=lax.Precision.DEFAULT) + bq_ref[...]
    k = jnp.dot(mem, wk_ref[...], preferred_element_type=jnp.float32,
                precision=lax.Precision.DEFAULT) + bk_ref[...]
    v = jnp.dot(mem, wv_ref[...], preferred_element_type=jnp.float32,
                precision=lax.Precision.DEFAULT) + bv_ref[...]
    s = jnp.sum(q * k, axis=-1, keepdims=True) * np.float32(1.0 / np.sqrt(D))
    g = jax.nn.sigmoid(s)
    out_ref[...] = jnp.dot(g * v, wo_ref[...], preferred_element_type=jnp.float32,
                           precision=lax.Precision.DEFAULT) + bo_ref[...]
    gw_ref[...] = g


def _make_dense(bs, blk, interpret=False):
    grid = (bs // blk,)
    Z = np.int32(0)
    full = lambda i: (Z, Z)
    row = lambda i: (i, Z)
    return pl.pallas_call(
        _dense_body,
        grid=grid,
        in_specs=[
            pl.BlockSpec((blk, D), row),
            pl.BlockSpec((blk, TOT), row),
            pl.BlockSpec((D, D), full),
            pl.BlockSpec((1, D), full),
            pl.BlockSpec((TOT, D), full),
            pl.BlockSpec((1, D), full),
            pl.BlockSpec((TOT, D), full),
            pl.BlockSpec((1, D), full),
            pl.BlockSpec((D, D), full),
            pl.BlockSpec((1, D), full),
        ],
        out_specs=[
            pl.BlockSpec((blk, D), row),
            pl.BlockSpec((blk, 1), row),
        ],
        out_shape=[
            jax.ShapeDtypeStruct((bs, D), jnp.float32),
            jax.ShapeDtypeStruct((bs, 1), jnp.float32),
        ],
        interpret=interpret,
    )


def kernel(token_ids, hidden_states, tables_n2, tables_n3,
           Wq, bq, Wk, bk, Wv, bv, Wo, bo):
    B, S = token_ids.shape
    bs = B * S
    n_pos = bs // NW

    tok = token_ids.astype(jnp.int32)
    t0 = tok.reshape(bs)
    tm1 = jnp.pad(tok, ((0, 0), (1, 0)))[:, :S].reshape(bs)
    tm2 = jnp.pad(tok, ((0, 0), (2, 0)))[:, :S].reshape(bs)

    tbl2v = jnp.transpose(tables_n2, (1, 0, 2))
    tbl3v = jnp.transpose(tables_n3, (1, 0, 2))
    mem2 = _make_sc_gather(bs, n_pos)(t0, tm1, tm2, tbl2v, tbl3v)

    hid = hidden_states.reshape(bs, D)
    out2, gw2 = _make_dense(bs, 512)(
        hid, mem2, Wq, bq.reshape(1, D), Wk, bk.reshape(1, D),
        Wv, bv.reshape(1, D), Wo, bo.reshape(1, D))

    return (mem2.reshape(B, S, TOT),
            out2.reshape(B, S, D).astype(jnp.float64),
            gw2.reshape(B, S, 1).astype(jnp.float64))
